# Initial kernel scaffold; baseline (speedup 1.0000x reference)
import jax
import jax.numpy as jnp
from jax.experimental import pallas as pl


def kernel(x, edge_index, Wq, Wk, Wv, Wskip, Wproj, bproj):
    raise NotImplementedError("write your pallas kernel here")


# speedup vs baseline: 6.3578x; 6.3578x over previous
import functools
import math

import jax
import jax.numpy as jnp
from jax import lax
from jax.experimental import pallas as pl
from jax.experimental.pallas import tpu as pltpu
from jax.experimental.pallas import tpu_sc as plsc

NC = 2
NS = 16
NW = NC * NS
L = 16

C = 80
CLAMP = 60.0


def _qkv_body(x_ref, w_ref, q_ref, kv_ref, xs_ref):
    y = jnp.dot(x_ref[...], w_ref[...], preferred_element_type=jnp.float32)
    d = q_ref.shape[-1]
    q_ref[...] = y[:, :d]
    kv_ref[...] = y[:, d:3 * d]
    xs_ref[...] = y[:, 3 * d:]


def _final_body(msg_ref, d0_ref, d1_ref, xs_ref, wp_ref, b_ref, out_ref):
    num = msg_ref[0] + msg_ref[1]
    den = d0_ref[...] + d1_ref[...] + 1e-16
    agg = num / den + xs_ref[...]
    out_ref[...] = jnp.dot(agg, wp_ref[...],
                           preferred_element_type=jnp.float32) + b_ref[...]


def _make_sc_kernel(n_nodes, n_edges, d):
    ept = n_edges // NW
    nchunk = ept // C
    assert ept % C == 0 and n_edges % NW == 0

    mesh = plsc.VectorSubcoreMesh(core_axis_name="c", subcore_axis_name="s",
                                  num_cores=NC, num_subcores=NS)

    @functools.partial(
        pl.kernel,
        out_type=[
            jax.ShapeDtypeStruct((NC, n_nodes, d), jnp.float32),
            jax.ShapeDtypeStruct((n_nodes,), jnp.float32),
            jax.ShapeDtypeStruct((n_nodes,), jnp.float32),
        ],
        mesh=mesh,
        scratch_types=[
            pltpu.VMEM((C,), jnp.int32),
            pltpu.VMEM((C,), jnp.int32),
            pltpu.VMEM((C, d), jnp.float32),
            pltpu.VMEM((C, 2 * d), jnp.float32),
            pltpu.VMEM((C, d), jnp.float32),
            pltpu.VMEM((C,), jnp.float32),
            pltpu.VMEM_SHARED((n_nodes, d), jnp.float32),
            pltpu.VMEM_SHARED((n_nodes,), jnp.float32),
            pltpu.SemaphoreType.DMA,
            pltpu.SemaphoreType.DMA,
        ],
        compiler_params=pltpu.CompilerParams(needs_layout_passes=False),
    )
    def sc_kernel(q_hbm, kv_hbm, src_hbm, dst_hbm, zmsg_hbm, zden_hbm,
                  msg_out, den0_out, den1_out,
                  src_v, dst_v, qrows, kvrows, orows, evals,
                  acc_msg, acc_den, sem_q, sem_kv):
        cid = lax.axis_index("c")
        sid = lax.axis_index("s")
        wid = sid * NC + cid

        @pl.when(sid == 0)
        def _init():
            pltpu.sync_copy(zmsg_hbm, acc_msg)
            pltpu.sync_copy(zden_hbm, acc_den)

        plsc.subcore_barrier()

        nblk = d // L

        def chunk_body(ci, _):
            off = wid * ept + ci * C
            pltpu.sync_copy(src_hbm.at[pl.ds(off, C)], src_v)
            pltpu.sync_copy(dst_hbm.at[pl.ds(off, C)], dst_v)
            cp_q = pltpu.async_copy(q_hbm.at[dst_v], qrows, sem_q)
            cp_kv = pltpu.async_copy(kv_hbm.at[src_v], kvrows, sem_kv)
            cp_q.wait()
            cp_kv.wait()

            lane = lax.iota(jnp.int32, L)

            def dot_body(g, _):
                base = g * L
                evec = jnp.zeros((L,), jnp.float32)
                for l in range(L):
                    i = base + l
                    acc = qrows[i, pl.ds(0, L)] * kvrows[i, pl.ds(0, L)]
                    for j in range(1, nblk):
                        acc += (qrows[i, pl.ds(j * L, L)]
                                * kvrows[i, pl.ds(j * L, L)])
                    evec = jnp.where(lane == l, jnp.sum(acc), evec)
                evals[pl.ds(base, L)] = jnp.exp(
                    jnp.clip(evec, -CLAMP, CLAMP))
                return ()

            lax.fori_loop(0, C // L, dot_body, ())

            def scale_body(i, _):
                eb = plsc.load_gather(evals, [jnp.full((L,), i, jnp.int32)])
                for j in range(nblk):
                    orows[i, pl.ds(j * L, L)] = (
                        kvrows[i, pl.ds(d + j * L, L)] * eb)
                return ()

            lax.fori_loop(0, C, scale_body, (), unroll=2)

            pltpu.sync_copy(orows, acc_msg.at[dst_v], add=True)
            pltpu.sync_copy(evals, acc_den.at[dst_v], add=True)
            return ()

        lax.fori_loop(0, nchunk, chunk_body, ())

        plsc.subcore_barrier()

        @pl.when(sid == 0)
        def _drain():
            pltpu.sync_copy(acc_msg, msg_out.at[cid])

        @pl.when((sid == 0) & (cid == 0))
        def _drain_den0():
            pltpu.sync_copy(acc_den, den0_out)

        @pl.when((sid == 0) & (cid == 1))
        def _drain_den1():
            pltpu.sync_copy(acc_den, den1_out)

    return sc_kernel


def kernel(x, edge_index, Wq, Wk, Wv, Wskip, Wproj, bproj):
    n, d_in = x.shape
    d = Wq.shape[1]
    dm = Wproj.shape[1]
    e = edge_index.shape[1]

    w_cat = jnp.concatenate(
        [Wq * jnp.float32(1.0 / math.sqrt(d)), Wk, Wv, Wskip], axis=1)

    blk = 1000 if n % 1000 == 0 else n
    grid = n // blk
    q, kv, xs = pl.pallas_call(
        _qkv_body,
        grid=(grid,),
        in_specs=[
            pl.BlockSpec((blk, d_in), lambda i: (i, 0)),
            pl.BlockSpec((d_in, 4 * d), lambda i: (0, 0)),
        ],
        out_specs=[
            pl.BlockSpec((blk, d), lambda i: (i, 0)),
            pl.BlockSpec((blk, 2 * d), lambda i: (i, 0)),
            pl.BlockSpec((blk, d), lambda i: (i, 0)),
        ],
        out_shape=[
            jax.ShapeDtypeStruct((n, d), jnp.float32),
            jax.ShapeDtypeStruct((n, 2 * d), jnp.float32),
            jax.ShapeDtypeStruct((n, d), jnp.float32),
        ],
    )(x, w_cat)

    zmsg = jnp.zeros((n, d), jnp.float32)
    zden = jnp.zeros((n,), jnp.float32)
    msg_p, den0, den1 = _make_sc_kernel(n, e, d)(
        q, kv, edge_index[0], edge_index[1], zmsg, zden)

    out = pl.pallas_call(
        _final_body,
        grid=(grid,),
        in_specs=[
            pl.BlockSpec((NC, blk, d), lambda i: (0, i, 0)),
            pl.BlockSpec((blk, 1), lambda i: (i, 0)),
            pl.BlockSpec((blk, 1), lambda i: (i, 0)),
            pl.BlockSpec((blk, d), lambda i: (i, 0)),
            pl.BlockSpec((d, dm), lambda i: (0, 0)),
            pl.BlockSpec((1, dm), lambda i: (0, 0)),
        ],
        out_specs=pl.BlockSpec((blk, dm), lambda i: (i, 0)),
        out_shape=jax.ShapeDtypeStruct((n, dm), jnp.float32),
    )(msg_p, den0.reshape(n, 1), den1.reshape(n, 1), xs, Wproj,
      bproj.reshape(1, dm))
    return out


# speedup vs baseline: 17.3257x; 2.7251x over previous
import functools
import math

import jax
import jax.numpy as jnp
from jax import lax
from jax.experimental import pallas as pl
from jax.experimental.pallas import tpu as pltpu
from jax.experimental.pallas import tpu_sc as plsc

NC = 2
NS = 16
NW = NC * NS
L = 16

C = 80
CLAMP = 60.0


def _qkv_body(x_ref, w_ref, q_ref, k_ref, v_ref, xs_ref):
    y = jnp.dot(x_ref[...], w_ref[...], preferred_element_type=jnp.float32)
    d = q_ref.shape[-1]
    q_ref[...] = y[:, :d]
    k_ref[...] = y[:, d:2 * d]
    v_ref[...] = y[:, 2 * d:3 * d]
    xs_ref[...] = y[:, 3 * d:]


def _final_body(msg_ref, d0_ref, d1_ref, xs_ref, wp_ref, b_ref, out_ref):
    num = msg_ref[0] + msg_ref[1]
    den = d0_ref[...] + d1_ref[...] + 1e-16
    agg = num / den + xs_ref[...]
    out_ref[...] = jnp.dot(agg, wp_ref[...],
                           preferred_element_type=jnp.float32) + b_ref[...]


def _make_sc_kernel(n_nodes, n_edges, d):
    ept = n_edges // NW
    nchunk = ept // C
    assert ept % C == 0 and n_edges % NW == 0 and C % L == 0

    mesh = plsc.VectorSubcoreMesh(core_axis_name="c", subcore_axis_name="s",
                                  num_cores=NC, num_subcores=NS)

    @functools.partial(
        pl.kernel,
        out_type=[
            jax.ShapeDtypeStruct((NC, n_nodes, d), jnp.float32),
            jax.ShapeDtypeStruct((n_nodes,), jnp.float32),
            jax.ShapeDtypeStruct((n_nodes,), jnp.float32),
        ],
        mesh=mesh,
        scratch_types=[
            pltpu.VMEM((C,), jnp.int32),
            pltpu.VMEM((C,), jnp.int32),
            pltpu.VMEM((C,), jnp.int32),
            pltpu.VMEM((C,), jnp.int32),
            pltpu.VMEM((C,), jnp.int32),
            pltpu.VMEM((C,), jnp.int32),
            pltpu.VMEM((C, d), jnp.float32),
            pltpu.VMEM((C, d), jnp.float32),
            pltpu.VMEM((C, d), jnp.float32),
            pltpu.VMEM((C, d), jnp.float32),
            pltpu.VMEM((C,), jnp.float32),
            pltpu.VMEM((C,), jnp.float32),
            pltpu.VMEM_SHARED((n_nodes, d), jnp.float32),
            pltpu.VMEM_SHARED((n_nodes,), jnp.float32),
            pltpu.SemaphoreType.DMA, pltpu.SemaphoreType.DMA,
            pltpu.SemaphoreType.DMA, pltpu.SemaphoreType.DMA,
            pltpu.SemaphoreType.DMA,
            pltpu.SemaphoreType.DMA,
            pltpu.SemaphoreType.DMA, pltpu.SemaphoreType.DMA,
            pltpu.SemaphoreType.DMA, pltpu.SemaphoreType.DMA,
            pltpu.SemaphoreType.DMA, pltpu.SemaphoreType.DMA,
        ],
        compiler_params=pltpu.CompilerParams(needs_layout_passes=False),
    )
    def sc_kernel(q_hbm, k_hbm, v_hbm, src_hbm, dst_hbm, zmsg_hbm, zden_hbm,
                  msg_out, den0_out, den1_out,
                  srcb0, srcb1, dstb0, dstb1, dsts0, dsts1,
                  qrows, krows, vrows0, vrows1, evals0, evals1,
                  acc_msg, acc_den,
                  sis0, sis1, sid0, sid1, sq, sk, sv0, sv1,
                  ssm0, ssm1, ssd0, ssd1):
        cid = lax.axis_index("c")
        sid = lax.axis_index("s")
        wid = sid * NC + cid
        base = wid * ept

        srcb = (srcb0, srcb1)
        dstb = (dstb0, dstb1)
        dsts = (dsts0, dsts1)
        vrows = (vrows0, vrows1)
        evals = (evals0, evals1)
        sis = (sis0, sis1)
        sidx = (sid0, sid1)
        sv = (sv0, sv1)
        ssm = (ssm0, ssm1)
        ssd = (ssd0, ssd1)

        @pl.when(sid == 0)
        def _init():
            pltpu.sync_copy(zmsg_hbm, acc_msg)
            pltpu.sync_copy(zden_hbm, acc_den)

        plsc.subcore_barrier()

        nblk = d // L
        lane = lax.iota(jnp.int32, L)

        def start_idx(ci, p):
            off = base + ci * C
            pltpu.async_copy(src_hbm.at[pl.ds(off, C)], srcb[p], sis[p])
            pltpu.async_copy(dst_hbm.at[pl.ds(off, C)], dstb[p], sidx[p])

        def wait_idx(p):
            pltpu.make_async_copy(src_hbm.at[pl.ds(0, C)], srcb[p],
                                  sis[p]).wait()
            pltpu.make_async_copy(dst_hbm.at[pl.ds(0, C)], dstb[p],
                                  sidx[p]).wait()

        def dot(p):
            def grp_body(g, _):
                gb = g * L

                def edge_body(l, evec):
                    i = gb + l
                    acc = qrows[i, pl.ds(0, L)] * krows[i, pl.ds(0, L)]
                    for j in range(1, nblk):
                        acc += (qrows[i, pl.ds(j * L, L)]
                                * krows[i, pl.ds(j * L, L)])
                    return jnp.where(lane == l, jnp.sum(acc), evec)

                evec = lax.fori_loop(0, L, edge_body,
                                     jnp.zeros((L,), jnp.float32))
                evals[p][pl.ds(gb, L)] = jnp.exp(
                    jnp.clip(evec, -CLAMP, CLAMP))
                return ()

            lax.fori_loop(0, C // L, grp_body, ())

        def scale(p):
            def scale_body(i, _):
                eb = plsc.load_gather(evals[p],
                                      [jnp.full((L,), i, jnp.int32)])
                for j in range(nblk):
                    sl = pl.ds(j * L, L)
                    vrows[p][i, sl] = vrows[p][i, sl] * eb
                return ()

            lax.fori_loop(0, C, scale_body, ())

        def maybe_when(cond, fn):
            if isinstance(cond, bool):
                if cond:
                    fn()
            else:
                pl.when(cond)(fn)

        pltpu.sync_copy(src_hbm.at[pl.ds(base, C)], srcb0)
        pltpu.sync_copy(dst_hbm.at[pl.ds(base, C)], dstb0)
        pltpu.async_copy(q_hbm.at[dstb0], qrows, sq)
        pltpu.async_copy(k_hbm.at[srcb0], krows, sk)
        pltpu.async_copy(v_hbm.at[srcb0], vrows0, sv0)
        start_idx(1, 1)

        def step(ci, b, last):
            p = b
            pltpu.make_async_copy(q_hbm.at[dstb[p]], qrows, sq).wait()
            pltpu.make_async_copy(k_hbm.at[srcb[p]], krows, sk).wait()
            maybe_when(ci >= 2, lambda: pltpu.make_async_copy(
                evals[p], acc_den.at[dsts[p]], ssd[p]).wait())
            dot(p)
            if not last:
                wait_idx(1 - p)
                pltpu.async_copy(q_hbm.at[dstb[1 - p]], qrows, sq)
                pltpu.async_copy(k_hbm.at[srcb[1 - p]], krows, sk)
                maybe_when(ci >= 1, lambda: pltpu.make_async_copy(
                    vrows[1 - p], acc_msg.at[dsts[1 - p]], ssm[1 - p]).wait())
                pltpu.async_copy(v_hbm.at[srcb[1 - p]], vrows[1 - p],
                                 sv[1 - p])

            def cp_body(m, _):
                sl = pl.ds(m * L, L)
                dsts[p][sl] = dstb[p][sl]
                return ()

            lax.fori_loop(0, C // L, cp_body, ())
            pltpu.make_async_copy(v_hbm.at[srcb[p]], vrows[p], sv[p]).wait()
            if not last:
                maybe_when(ci + 2 < nchunk, lambda: start_idx(ci + 2, p))
            scale(p)
            pltpu.async_copy(vrows[p], acc_msg.at[dsts[p]], ssm[p], add=True)
            pltpu.async_copy(evals[p], acc_den.at[dsts[p]], ssd[p], add=True)

        def pipe_body(g, _):
            step(2 * g, 0, False)
            step(2 * g + 1, 1, False)
            return ()

        if nchunk % 2 == 1:
            lax.fori_loop(0, nchunk // 2, pipe_body, ())
            step(nchunk - 1, 0, True)
        else:
            lax.fori_loop(0, nchunk // 2 - 1, pipe_body, ())
            step(nchunk - 2, 0, False)
            step(nchunk - 1, 1, True)

        pltpu.make_async_copy(vrows0, acc_msg.at[dsts0], ssm0).wait()
        pltpu.make_async_copy(evals0, acc_den.at[dsts0], ssd0).wait()
        pltpu.make_async_copy(vrows1, acc_msg.at[dsts1], ssm1).wait()
        pltpu.make_async_copy(evals1, acc_den.at[dsts1], ssd1).wait()

        plsc.subcore_barrier()

        @pl.when(sid == 0)
        def _drain():
            pltpu.sync_copy(acc_msg, msg_out.at[cid])

        @pl.when((sid == 0) & (cid == 0))
        def _drain_den0():
            pltpu.sync_copy(acc_den, den0_out)

        @pl.when((sid == 0) & (cid == 1))
        def _drain_den1():
            pltpu.sync_copy(acc_den, den1_out)

    return sc_kernel


def kernel(x, edge_index, Wq, Wk, Wv, Wskip, Wproj, bproj):
    n, d_in = x.shape
    d = Wq.shape[1]
    dm = Wproj.shape[1]
    e = edge_index.shape[1]

    w_cat = jnp.concatenate(
        [Wq * jnp.float32(1.0 / math.sqrt(d)), Wk, Wv, Wskip], axis=1)

    blk = 1000 if n % 1000 == 0 else n
    grid = n // blk
    q, k, v, xs = pl.pallas_call(
        _qkv_body,
        grid=(grid,),
        in_specs=[
            pl.BlockSpec((blk, d_in), lambda i: (i, 0)),
            pl.BlockSpec((d_in, 4 * d), lambda i: (0, 0)),
        ],
        out_specs=[
            pl.BlockSpec((blk, d), lambda i: (i, 0)),
            pl.BlockSpec((blk, d), lambda i: (i, 0)),
            pl.BlockSpec((blk, d), lambda i: (i, 0)),
            pl.BlockSpec((blk, d), lambda i: (i, 0)),
        ],
        out_shape=[
            jax.ShapeDtypeStruct((n, d), jnp.float32),
            jax.ShapeDtypeStruct((n, d), jnp.float32),
            jax.ShapeDtypeStruct((n, d), jnp.float32),
            jax.ShapeDtypeStruct((n, d), jnp.float32),
        ],
    )(x, w_cat)

    zmsg = jnp.zeros((n, d), jnp.float32)
    zden = jnp.zeros((n,), jnp.float32)
    msg_p, den0, den1 = _make_sc_kernel(n, e, d)(
        q, k, v, edge_index[0], edge_index[1], zmsg, zden)

    out = pl.pallas_call(
        _final_body,
        grid=(grid,),
        in_specs=[
            pl.BlockSpec((NC, blk, d), lambda i: (0, i, 0)),
            pl.BlockSpec((blk, 1), lambda i: (i, 0)),
            pl.BlockSpec((blk, 1), lambda i: (i, 0)),
            pl.BlockSpec((blk, d), lambda i: (i, 0)),
            pl.BlockSpec((d, dm), lambda i: (0, 0)),
            pl.BlockSpec((1, dm), lambda i: (0, 0)),
        ],
        out_specs=pl.BlockSpec((blk, dm), lambda i: (i, 0)),
        out_shape=jax.ShapeDtypeStruct((n, dm), jnp.float32),
    )(msg_p, den0.reshape(n, 1), den1.reshape(n, 1), xs, Wproj,
      bproj.reshape(1, dm))
    return out
